# baseline (speedup 1.0000x reference)
import functools

import jax
import jax.numpy as jnp
from jax import lax
from jax.experimental import pallas as pl
from jax.experimental.pallas import tpu as pltpu
from jax.experimental.pallas import tpu_sc as plsc

_NC = 2
_NS = 16
_NW = _NC * _NS
_W = 800
_NBUF = 4


def kernel(table, article_indices):
    batch, hist = article_indices.shape
    num_idx = batch * hist
    embed = table.shape[1]
    idx = article_indices.reshape(num_idx).astype(jnp.int32)

    b_per_w = num_idx // _NW
    n_blocks = b_per_w // _W
    n_rounds = n_blocks // _NBUF
    max_off = num_idx - _W
    bat_per_blk = _W // hist

    mesh = plsc.VectorSubcoreMesh(core_axis_name="c", subcore_axis_name="s")

    scratch = (
        [pltpu.VMEM((_W,), jnp.int32) for _ in range(_NBUF)]
        + [pltpu.VMEM((_W, embed), jnp.float32) for _ in range(_NBUF)]
        + [pltpu.SemaphoreType.DMA for _ in range(3 * _NBUF)]
    )

    @functools.partial(
        pl.kernel,
        mesh=mesh,
        out_type=jax.ShapeDtypeStruct((batch, hist, embed), table.dtype),
        scratch_types=scratch,
        compiler_params=pltpu.CompilerParams(use_tc_tiling_on_sc=False),
    )
    def gather_kernel(table_hbm, idx_hbm, out_hbm, *bufs):
        idx_v = bufs[:_NBUF]
        rows_v = bufs[_NBUF:2 * _NBUF]
        sem_i = bufs[2 * _NBUF:3 * _NBUF]
        sem_g = bufs[3 * _NBUF:4 * _NBUF]
        sem_o = bufs[4 * _NBUF:5 * _NBUF]

        wid = lax.axis_index("s") * _NC + lax.axis_index("c")
        base = wid * b_per_w
        bat0 = wid * (b_per_w // hist)

        def idx_off(blk):
            return jnp.minimum(base + blk * _W, max_off)

        def fetch_idx(b, blk):
            pltpu.async_copy(
                idx_hbm.at[pl.ds(idx_off(blk), _W)], idx_v[b], sem_i[b]
            )

        def fire(b):
            pltpu.async_copy(table_hbm.at[idx_v[b]], rows_v[b], sem_g[b])

        def drain_writeback(b, blk):
            for k in range(bat_per_blk):
                pltpu.async_copy(
                    rows_v[b].at[pl.ds(k * hist, hist)],
                    out_hbm.at[bat0 + blk * bat_per_blk + k],
                    sem_o[b],
                )

        def wait_idx(b):
            pltpu.make_async_copy(
                idx_hbm.at[pl.ds(0, _W)], idx_v[b], sem_i[b]
            ).wait()

        def wait_gather(b):
            pltpu.make_async_copy(
                table_hbm.at[pl.ds(0, _W)], rows_v[b], sem_g[b]
            ).wait()

        def wait_out(b):
            for _ in range(bat_per_blk):
                pltpu.make_async_copy(
                    rows_v[b].at[pl.ds(0, hist)], out_hbm.at[0], sem_o[b]
                ).wait()

        for b in range(_NBUF):
            fetch_idx(b, b)

        for b in range(_NBUF):
            wait_idx(b)
            fire(b)
        for b in range(_NBUF):
            wait_gather(b)
            drain_writeback(b, b)
            fetch_idx(b, _NBUF + b)

        @pl.loop(1, n_rounds)
        def _(r):
            blk0 = r * _NBUF
            for b in range(_NBUF):
                wait_idx(b)
                wait_out(b)
                fire(b)
            for b in range(_NBUF):
                wait_gather(b)
                drain_writeback(b, blk0 + b)
                fetch_idx(b, blk0 + _NBUF + b)

        for b in range(_NBUF):
            wait_out(b)
            wait_idx(b)

    return gather_kernel(table, idx)
